# baseline (speedup 1.0000x reference)
import jax
import jax.numpy as jnp
from jax.experimental import pallas as pl
from jax.experimental.pallas import tpu as pltpu
from jax.experimental.pallas import tpu_sc as plsc

N_ATOMS = 4096
N_CGS = 512
KNN = 32
B_BATCH = 4
LANES = 16

BN1 = 512
BN3 = 128
BN4 = 512


def _k1_softmax_stats(logits_ref, xyzc_ref, bcast_ref, colsum_ref, gtun_ref,
                      idx_ref, d3_ref):
    i = pl.program_id(0)
    x = logits_ref[...]
    m = jnp.max(x, axis=1, keepdims=True)
    e = jnp.exp(x - m)
    s = jnp.sum(e, axis=1, keepdims=True)
    soft = e / s
    bcast_ref[...] = jnp.broadcast_to(soft[None], (B_BATCH, BN1, N_CGS))

    col = jax.lax.broadcasted_iota(jnp.int32, (BN1, N_CGS), 1)
    hit = jnp.where(x == m, col, N_CGS)
    idx_ref[...] = jnp.min(hit, axis=1, keepdims=True)

    softT = jnp.transpose(soft)
    part_cs = jnp.sum(softT, axis=1, keepdims=True)
    part_gt = jnp.dot(softT, xyzc_ref[...],
                      preferred_element_type=jnp.float32)

    @pl.when(i == 0)
    def _():
        colsum_ref[...] = part_cs
        gtun_ref[...] = part_gt

    @pl.when(i != 0)
    def _():
        colsum_ref[...] += part_cs
        gtun_ref[...] += part_gt

    @pl.when(i == pl.num_programs(0) - 1)
    def _():
        r = 1.0 / (colsum_ref[...] + 1e-8)
        gt = gtun_ref[...] * r
        g1 = jax.lax.slice(gt, (1, 0), (KNN + 1, LANES))
        d3_ref[...] = g1[None, :, :] - gt[:, None, :]


def _k2_big_matmul(b_ref, d_ref, soft_ref, gtun_ref, colsum_ref,
                   dx_ref, vt_ref, tbl_ref):
    i = pl.program_id(0)
    dx = jnp.dot(b_ref[...], d_ref[...],
                 preferred_element_type=jnp.float32)
    dx_ref[...] = dx
    softT = jnp.transpose(soft_ref[0])
    part = jnp.dot(softT, dx, preferred_element_type=jnp.float32)

    @pl.when(i == 0)
    def _():
        vt_ref[...] = part

    @pl.when(i != 0)
    def _():
        vt_ref[...] += part

    @pl.when(i == pl.num_programs(0) - 1)
    def _():
        r = 1.0 / (colsum_ref[...] + 1e-8)
        val = (gtun_ref[...] - vt_ref[...]) * r
        tbl_ref[...] = jnp.concatenate(
            [val, jnp.zeros((N_CGS, 128 - LANES), jnp.float32)], axis=1)


GWIN = 128


def _sc_gather_combine(tbl, idx_row, dx_all):
    mesh = plsc.VectorSubcoreMesh(core_axis_name="c", subcore_axis_name="s")

    @pl.kernel(
        out_type=jax.ShapeDtypeStruct((N_ATOMS, LANES), jnp.float32),
        mesh=mesh,
        scratch_types=[pltpu.VMEM((GWIN, 128), jnp.float32)])
    def sc_kernel(tbl_hbm, idx_hbm, dx_hbm, out_hbm, t128_ref):
        def body(i_vmem, dx_vmem, o_vmem):
            pltpu.sync_copy(tbl_hbm.at[i_vmem.at[0]], t128_ref)

            @pl.loop(0, GWIN)
            def _(a):
                sl = (pl.ds(a, 1), pl.ds(0, LANES))
                o_vmem.at[sl][...] = t128_ref.at[sl][...] + dx_vmem.at[sl][...]

        pltpu.emit_pipeline(
            body,
            grid=(N_ATOMS // GWIN,),
            in_specs=[
                pl.BlockSpec((1, GWIN), lambda i: (0, i)),
                pl.BlockSpec((GWIN, LANES), lambda i: (i, 0)),
            ],
            out_specs=[pl.BlockSpec((GWIN, LANES), lambda i: (i, 0))],
            core_axis_name=("c", "s"),
            dimension_semantics=(pltpu.PARALLEL,),
        )(idx_hbm, dx_hbm, out_hbm)

    return sc_kernel(tbl, idx_row, dx_all)


def kernel(xyz, z, nbr_list, bonds, assign_logits, B_param):
    f32 = jnp.float32

    xyzc = jnp.pad(jnp.transpose(xyz, (1, 0, 2)),
                   ((0, 0), (0, 0), (0, 1))).reshape(N_ATOMS, LANES)

    grid1 = N_ATOMS // BN1
    soft_bcast, colsum, gt_un, idx_col, d3 = pl.pallas_call(
        _k1_softmax_stats,
        grid=(grid1,),
        in_specs=[
            pl.BlockSpec((BN1, N_CGS), lambda i: (i, 0)),
            pl.BlockSpec((BN1, LANES), lambda i: (i, 0)),
        ],
        out_specs=[
            pl.BlockSpec((B_BATCH, BN1, N_CGS), lambda i: (0, i, 0)),
            pl.BlockSpec((N_CGS, 1), lambda i: (0, 0)),
            pl.BlockSpec((N_CGS, LANES), lambda i: (0, 0)),
            pl.BlockSpec((BN1, 1), lambda i: (i, 0)),
            pl.BlockSpec((N_CGS, KNN, LANES), lambda i: (0, 0, 0)),
        ],
        out_shape=[
            jax.ShapeDtypeStruct((B_BATCH, N_ATOMS, N_CGS), f32),
            jax.ShapeDtypeStruct((N_CGS, 1), f32),
            jax.ShapeDtypeStruct((N_CGS, LANES), f32),
            jax.ShapeDtypeStruct((N_ATOMS, 1), jnp.int32),
            jax.ShapeDtypeStruct((N_CGS, KNN, LANES), f32),
        ],
    )(assign_logits, xyzc)

    d_flat = d3.reshape(N_CGS * KNN, LANES)

    grid3 = N_ATOMS // BN3
    dx_all, vt, tbl = pl.pallas_call(
        _k2_big_matmul,
        grid=(grid3,),
        in_specs=[
            pl.BlockSpec((BN3, N_CGS * KNN), lambda i: (i, 0)),
            pl.BlockSpec((N_CGS * KNN, LANES), lambda i: (0, 0)),
            pl.BlockSpec((1, BN3, N_CGS), lambda i: (0, i, 0)),
            pl.BlockSpec((N_CGS, LANES), lambda i: (0, 0)),
            pl.BlockSpec((N_CGS, 1), lambda i: (0, 0)),
        ],
        out_specs=[
            pl.BlockSpec((BN3, LANES), lambda i: (i, 0)),
            pl.BlockSpec((N_CGS, LANES), lambda i: (0, 0)),
            pl.BlockSpec((N_CGS, 128), lambda i: (0, 0)),
        ],
        out_shape=[
            jax.ShapeDtypeStruct((N_ATOMS, LANES), f32),
            jax.ShapeDtypeStruct((N_CGS, LANES), f32),
            jax.ShapeDtypeStruct((N_CGS, 128), f32),
        ],
    )(B_param, d_flat, soft_bcast, gt_un, colsum)

    idx_row = idx_col.reshape(1, N_ATOMS)
    recon16 = _sc_gather_combine(tbl, idx_row, dx_all)

    xyz_recon = jnp.transpose(
        recon16.reshape(N_ATOMS, B_BATCH, 4), (1, 0, 2))[:, :, :3]
    return (soft_bcast, xyz, xyz_recon)
